# Initial kernel scaffold; baseline (speedup 1.0000x reference)
import jax
import jax.numpy as jnp
from jax.experimental import pallas as pl


def kernel(inputs, targets):
    raise NotImplementedError("write your pallas kernel here")


# speedup vs baseline: 2.0924x; 2.0924x over previous
import functools

import jax
import jax.numpy as jnp
from jax import lax
from jax.experimental import pallas as pl
from jax.experimental.pallas import tpu as pltpu
from jax.experimental.pallas import tpu_sc as plsc

_NC = 2
_NS = 16
_NW = _NC * _NS
_L = 16
_LANES = 128


def _sc_gather(flat_in, tgt2d, C):
    R = tgt2d.shape[0]
    RW = R // _NW
    mesh = plsc.VectorSubcoreMesh(
        core_axis_name="c", subcore_axis_name="s",
        num_cores=_NC, num_subcores=_NS,
    )

    @functools.partial(
        pl.kernel,
        out_type=jax.ShapeDtypeStruct((R, _LANES), jnp.float32),
        mesh=mesh,
        scratch_types=[
            pltpu.VMEM((RW, _LANES), jnp.int32),
            pltpu.VMEM((RW, _LANES), jnp.int32),
            pltpu.VMEM((RW, _LANES), jnp.float32),
            pltpu.SemaphoreType.DMA,
        ],
    )
    def gather_kernel(in_hbm, tgt_hbm, out_hbm, tgt_v, idx_v, p_v, sem):
        wid = lax.axis_index("s") * _NC + lax.axis_index("c")
        row0 = wid * RW
        pltpu.sync_copy(tgt_hbm.at[pl.ds(row0, RW)], tgt_v)
        lane = lax.iota(jnp.int32, _L)
        for j in range(RW):
            for k in range(_LANES // _L):
                t = tgt_v[j, pl.ds(k * _L, _L)]
                base = (row0 + j) * _LANES + k * _L
                idx_v[j, pl.ds(k * _L, _L)] = (base + lane) * C + t
        copies = [
            pltpu.async_copy(in_hbm.at[idx_v.at[j]], p_v.at[j], sem)
            for j in range(RW)
        ]
        for cp in copies:
            cp.wait()
        pltpu.sync_copy(p_v, out_hbm.at[pl.ds(row0, RW)])

    return gather_kernel(flat_in, tgt2d)


def _tc_loss_mean(p2d):
    n = p2d.shape[0] * p2d.shape[1]

    def body(p_ref, o_ref):
        p = p_ref[...]
        loss = jnp.log(p) - jnp.square(1.0 - p)
        o_ref[0, 0] = jnp.sum(loss) * (1.0 / n)

    return pl.pallas_call(
        body,
        out_shape=jax.ShapeDtypeStruct((1, 1), jnp.float32),
        out_specs=pl.BlockSpec(memory_space=pltpu.SMEM),
    )(p2d)


def kernel(inputs, targets):
    N, C = inputs.shape
    flat = inputs.reshape(N * C)
    tgt2d = targets.astype(jnp.int32).reshape(N // _LANES, _LANES)
    p2d = _sc_gather(flat, tgt2d, C)
    return _tc_loss_mean(p2d)[0, 0]


# speedup vs baseline: 3.0134x; 1.4401x over previous
import functools

import jax
import jax.numpy as jnp
from jax import lax
from jax.experimental import pallas as pl
from jax.experimental.pallas import tpu as pltpu
from jax.experimental.pallas import tpu_sc as plsc

_NC = 2
_NS = 16
_NW = _NC * _NS
_L = 16
_CHUNK = 64


def _sc_gather(in2d, tgt1d):
    N, C = in2d.shape
    RW = N // _NW
    mesh = plsc.VectorSubcoreMesh(
        core_axis_name="c", subcore_axis_name="s",
        num_cores=_NC, num_subcores=_NS,
    )

    @functools.partial(
        pl.kernel,
        out_type=jax.ShapeDtypeStruct((N,), jnp.float32),
        mesh=mesh,
        scratch_types=[
            pltpu.VMEM((RW,), jnp.int32),
            pltpu.VMEM((RW,), jnp.float32),
            pltpu.VMEM((_CHUNK, C), jnp.float32),
        ],
        compiler_params=pltpu.CompilerParams(
            use_tc_tiling_on_sc=True, needs_layout_passes=False,
        ),
    )
    def gather_kernel(in_hbm, tgt_hbm, out_hbm, tgt_v, p_v, buf):
        wid = lax.axis_index("s") * _NC + lax.axis_index("c")
        row0 = wid * RW
        pltpu.sync_copy(tgt_hbm.at[pl.ds(row0, RW)], tgt_v)
        lane = lax.iota(jnp.int32, _L)
        for k in range(RW // _CHUNK):
            pltpu.sync_copy(in_hbm.at[pl.ds(row0 + k * _CHUNK, _CHUNK)], buf)
            for g in range(_CHUNK // _L):
                off = k * _CHUNK + g * _L
                t = tgt_v[pl.ds(off, _L)]
                lrow = g * _L + lane
                p_v[pl.ds(off, _L)] = plsc.load_gather(buf, [lrow, t])
        pltpu.sync_copy(p_v, out_hbm.at[pl.ds(row0, RW)])

    return gather_kernel(in2d, tgt1d)


def _tc_loss_mean(p2d):
    n = p2d.shape[0] * p2d.shape[1]

    def body(p_ref, o_ref):
        p = p_ref[...]
        loss = jnp.log(p) - jnp.square(1.0 - p)
        o_ref[0, 0] = jnp.sum(loss) * (1.0 / n)

    return pl.pallas_call(
        body,
        out_shape=jax.ShapeDtypeStruct((1, 1), jnp.float32),
        out_specs=pl.BlockSpec(memory_space=pltpu.SMEM),
    )(p2d)


def kernel(inputs, targets):
    N, C = inputs.shape
    tgt1d = targets.astype(jnp.int32).reshape(N)
    p1d = _sc_gather(inputs, tgt1d)
    return _tc_loss_mean(p1d.reshape(N // 128, 128))[0, 0]


# speedup vs baseline: 10.4230x; 3.4589x over previous
import functools

import jax
import jax.numpy as jnp
from jax import lax
from jax.experimental import pallas as pl
from jax.experimental.pallas import tpu as pltpu
from jax.experimental.pallas import tpu_sc as plsc

_NC = 2
_NS = 16
_NW = _NC * _NS
_L = 16


def _sc_gather(in_t, tgt1d):
    C, N = in_t.shape
    CW = N // _NW
    W = 128
    NQ = CW // W
    mesh = plsc.VectorSubcoreMesh(
        core_axis_name="c", subcore_axis_name="s",
        num_cores=_NC, num_subcores=_NS,
    )

    @functools.partial(
        pl.kernel,
        out_type=jax.ShapeDtypeStruct((N,), jnp.float32),
        mesh=mesh,
        scratch_types=[
            pltpu.VMEM((CW,), jnp.int32),
            pltpu.VMEM((CW,), jnp.float32),
            pltpu.VMEM((NQ, W, W), jnp.float32),
            pltpu.SemaphoreType.DMA,
        ],
        compiler_params=pltpu.CompilerParams(
            use_tc_tiling_on_sc=True, needs_layout_passes=False,
        ),
    )
    def gather_kernel(in_hbm, tgt_hbm, out_hbm, tgt_v, p_v, win_v, sem):
        wid = lax.axis_index("s") * _NC + lax.axis_index("c")
        col0 = wid * CW
        pltpu.sync_copy(tgt_hbm.at[pl.ds(col0, CW)], tgt_v)
        copies = []
        for q in range(NQ):
            rows = tgt_v.at[pl.ds(q * W, W)]
            copies.append(pltpu.async_copy(
                in_hbm.at[rows, pl.ds(col0 + q * W, W)], win_v.at[q], sem))
        for cp in copies:
            cp.wait()
        lane = lax.iota(jnp.int32, _L)
        for q in range(NQ):
            qv = jnp.full((_L,), q, jnp.int32)
            for g in range(W // _L):
                d = g * _L + lane
                p_v[pl.ds(q * W + g * _L, _L)] = plsc.load_gather(
                    win_v, [qv, d, d])
        pltpu.sync_copy(p_v, out_hbm.at[pl.ds(col0, CW)])

    return gather_kernel(in_t, tgt1d)


def _tc_loss_mean(p2d):
    n = p2d.shape[0] * p2d.shape[1]

    def body(p_ref, o_ref):
        p = p_ref[...]
        loss = jnp.log(p) - jnp.square(1.0 - p)
        o_ref[0, 0] = jnp.sum(loss) * (1.0 / n)

    return pl.pallas_call(
        body,
        out_shape=jax.ShapeDtypeStruct((1, 1), jnp.float32),
        out_specs=pl.BlockSpec(memory_space=pltpu.SMEM),
    )(p2d)


def kernel(inputs, targets):
    N, C = inputs.shape
    tgt1d = targets.astype(jnp.int32).reshape(N)
    p1d = _sc_gather(inputs.T, tgt1d)
    return _tc_loss_mean(p1d.reshape(N // 128, 128))[0, 0]


# speedup vs baseline: 11.0290x; 1.0581x over previous
import functools

import jax
import jax.numpy as jnp
from jax import lax
from jax.experimental import pallas as pl
from jax.experimental.pallas import tpu as pltpu
from jax.experimental.pallas import tpu_sc as plsc

_NC = 2
_NS = 16
_NW = _NC * _NS
_L = 16
_LN2 = 0.6931471805599453
_SQRT2 = 1.4142135623730951


def _log_f32(p):
    bits = plsc.bitcast(p, jnp.int32)
    e = (bits >> 23) - 127
    m = plsc.bitcast((bits & 0x7FFFFF) | 0x3F800000, jnp.float32)
    big = m > _SQRT2
    m = jnp.where(big, m * 0.5, m)
    e = jnp.where(big, e + 1, e)
    s = (m - 1.0) / (m + 1.0)
    s2 = s * s
    ln_m = 2.0 * s * (1.0 + s2 * (1.0 / 3.0 + s2 * (1.0 / 5.0 + s2 * (1.0 / 7.0 + s2 * (1.0 / 9.0)))))
    ln_p = ln_m + e.astype(jnp.float32) * _LN2
    return jnp.where(p == 0.0, -jnp.inf, ln_p)


def _sc_loss_partials(in_t, tgt1d):
    C, N = in_t.shape
    CW = N // _NW
    W = 128
    NQ = CW // W
    mesh = plsc.VectorSubcoreMesh(
        core_axis_name="c", subcore_axis_name="s",
        num_cores=_NC, num_subcores=_NS,
    )

    @functools.partial(
        pl.kernel,
        out_type=jax.ShapeDtypeStruct((_NW * _L,), jnp.float32),
        mesh=mesh,
        scratch_types=[
            pltpu.VMEM((CW,), jnp.int32),
            pltpu.VMEM((_L,), jnp.float32),
            pltpu.VMEM((NQ, W, W), jnp.float32),
            pltpu.SemaphoreType.DMA,
        ],
        compiler_params=pltpu.CompilerParams(
            use_tc_tiling_on_sc=True, needs_layout_passes=False,
        ),
    )
    def loss_kernel(in_hbm, tgt_hbm, out_hbm, tgt_v, acc_v, win_v, sem):
        wid = lax.axis_index("s") * _NC + lax.axis_index("c")
        col0 = wid * CW
        pltpu.sync_copy(tgt_hbm.at[pl.ds(col0, CW)], tgt_v)
        copies = []
        for q in range(NQ):
            rows = tgt_v.at[pl.ds(q * W, W)]
            copies.append(pltpu.async_copy(
                in_hbm.at[rows, pl.ds(col0 + q * W, W)], win_v.at[q], sem))
        for cp in copies:
            cp.wait()
        lane = lax.iota(jnp.int32, _L)

        def body(i, acc):
            q = i >> 3
            d = (i & 7) * _L + lane
            p = plsc.load_gather(win_v, [jnp.full((_L,), 0, jnp.int32) + q, d, d])
            r = 1.0 - p
            return acc + (_log_f32(p) - r * r)

        acc = lax.fori_loop(0, NQ * (W // _L), body, jnp.zeros((_L,), jnp.float32))
        acc_v[...] = acc
        pltpu.sync_copy(acc_v, out_hbm.at[pl.ds(wid * _L, _L)])

    return loss_kernel(in_t, tgt1d)


def _tc_mean(partials2d, n):
    def body(p_ref, o_ref):
        o_ref[0, 0] = jnp.sum(p_ref[...]) * (1.0 / n)

    return pl.pallas_call(
        body,
        out_shape=jax.ShapeDtypeStruct((1, 1), jnp.float32),
        out_specs=pl.BlockSpec(memory_space=pltpu.SMEM),
    )(partials2d)


def kernel(inputs, targets):
    N, C = inputs.shape
    tgt1d = targets.astype(jnp.int32).reshape(N)
    part = _sc_loss_partials(inputs.T, tgt1d)
    return _tc_mean(part.reshape(4, 128), N)[0, 0]


# speedup vs baseline: 11.1133x; 1.0076x over previous
import functools

import jax
import jax.numpy as jnp
from jax import lax
from jax.experimental import pallas as pl
from jax.experimental.pallas import tpu as pltpu
from jax.experimental.pallas import tpu_sc as plsc

_NC = 2
_NS = 16
_NW = _NC * _NS
_L = 16
_LN2 = 0.6931471805599453
_SQRT2 = 1.4142135623730951


def _log_f32(p):
    bits = plsc.bitcast(p, jnp.int32)
    e = (bits >> 23) - 127
    m = plsc.bitcast((bits & 0x7FFFFF) | 0x3F800000, jnp.float32)
    big = m > _SQRT2
    m = jnp.where(big, m * 0.5, m)
    e = jnp.where(big, e + 1, e)
    s = (m - 1.0) / (m + 1.0)
    s2 = s * s
    ln_m = 2.0 * s * (1.0 + s2 * (1.0 / 3.0 + s2 * (1.0 / 5.0 + s2 * (1.0 / 7.0 + s2 * (1.0 / 9.0)))))
    ln_p = ln_m + e.astype(jnp.float32) * _LN2
    return jnp.where(p == 0.0, -jnp.inf, ln_p)


def _sc_loss_partials(in_t, tgt1d):
    C, N = in_t.shape
    CW = N // _NW
    W = 128
    NQ = CW // W
    mesh = plsc.VectorSubcoreMesh(
        core_axis_name="c", subcore_axis_name="s",
        num_cores=_NC, num_subcores=_NS,
    )

    @functools.partial(
        pl.kernel,
        out_type=jax.ShapeDtypeStruct((_NW * _L,), jnp.float32),
        mesh=mesh,
        scratch_types=[
            pltpu.VMEM((CW,), jnp.int32),
            pltpu.VMEM((_L,), jnp.float32),
            pltpu.VMEM((NQ, W, W), jnp.float32),
            pltpu.SemaphoreType.DMA,
        ],
        compiler_params=pltpu.CompilerParams(
            use_tc_tiling_on_sc=True, needs_layout_passes=False,
            skip_device_barrier=True,
        ),
    )
    def loss_kernel(in_hbm, tgt_hbm, out_hbm, tgt_v, acc_v, win_v, sem):
        wid = lax.axis_index("s") * _NC + lax.axis_index("c")
        col0 = wid * CW
        pltpu.sync_copy(tgt_hbm.at[pl.ds(col0, CW)], tgt_v)
        copies = []
        for q in range(NQ):
            rows = tgt_v.at[pl.ds(q * W, W)]
            copies.append(pltpu.async_copy(
                in_hbm.at[rows, pl.ds(col0 + q * W, W)], win_v.at[q], sem))
        for cp in copies:
            cp.wait()
        lane = lax.iota(jnp.int32, _L)

        def body(i, acc):
            q = i >> 3
            d = (i & 7) * _L + lane
            p = plsc.load_gather(win_v, [jnp.full((_L,), 0, jnp.int32) + q, d, d])
            r = 1.0 - p
            return acc + (_log_f32(p) - r * r)

        acc = lax.fori_loop(0, NQ * (W // _L), body, jnp.zeros((_L,), jnp.float32))
        acc_v[...] = acc
        pltpu.sync_copy(acc_v, out_hbm.at[pl.ds(wid * _L, _L)])

    return loss_kernel(in_t, tgt1d)


def _tc_mean(partials2d, n):
    def body(p_ref, o_ref):
        o_ref[0, 0] = jnp.sum(p_ref[...]) * (1.0 / n)

    return pl.pallas_call(
        body,
        out_shape=jax.ShapeDtypeStruct((1, 1), jnp.float32),
        out_specs=pl.BlockSpec(memory_space=pltpu.SMEM),
        compiler_params=pltpu.CompilerParams(skip_device_barrier=True),
    )(partials2d)


def kernel(inputs, targets):
    N, C = inputs.shape
    tgt1d = targets.astype(jnp.int32).reshape(N)
    part = _sc_loss_partials(inputs.T, tgt1d)
    return _tc_mean(part.reshape(4, 128), N)[0, 0]
